# baseline (speedup 1.0000x reference)
import functools

import jax
import jax.numpy as jnp
from jax import lax
from jax.experimental import pallas as pl
from jax.experimental.pallas import tpu as pltpu
from jax.experimental.pallas import tpu_sc as plsc

N, U, V = 16, 512, 1024
P = 0.5
NB = 2

R = N * U
NC, NS = 2, 16
NW = NC * NS
RPW = R // NW
ZR = 32


def _tc_body(y_ref, pf_ref, lp_ref, loss_ref, kl_ref):
    y = y_ref[:, 0, :]
    lp = lp_ref[...]
    pf = pf_ref[...]
    ycol = y[:, :, None]
    iota = lax.broadcasted_iota(jnp.int32, (NB, U, V), 2)
    hit = (iota == ycol) & (ycol != 0)
    eye = jnp.where(hit, jnp.float32(1.0 - P), jnp.float32(0.0))
    kl = jnp.float32(P) * (pf * lp)
    kl_ref[...] = kl
    loss_ref[...] = eye * lp + kl


def _tc_call(y3, p_fixed, logp_train):
    out_shape = jax.ShapeDtypeStruct((N, U, V), jnp.float32)
    return pl.pallas_call(
        _tc_body,
        grid=(N // NB,),
        in_specs=[
            pl.BlockSpec((NB, 1, U), lambda n: (n, 0, 0)),
            pl.BlockSpec((NB, U, V), lambda n: (n, 0, 0)),
            pl.BlockSpec((NB, U, V), lambda n: (n, 0, 0)),
        ],
        out_specs=[
            pl.BlockSpec((NB, U, V), lambda n: (n, 0, 0)),
            pl.BlockSpec((NB, U, V), lambda n: (n, 0, 0)),
        ],
        out_shape=[out_shape, out_shape],
    )(y3, p_fixed, logp_train)


NG = RPW // 16
NCHUNK = RPW // ZR
GPC = ZR // 16


@functools.partial(
    pl.kernel,
    mesh=plsc.VectorSubcoreMesh(core_axis_name="c", subcore_axis_name="s"),
    out_type=jax.ShapeDtypeStruct((R * V,), jnp.float32),
    scratch_types=[
        pltpu.VMEM((RPW,), jnp.int32),
        pltpu.VMEM((NG, 16), jnp.int32),
        pltpu.VMEM((NG, 16), jnp.float32),
        pltpu.VMEM((ZR * V,), jnp.float32),
        pltpu.VMEM((ZR * V,), jnp.float32),
        pltpu.SemaphoreType.DMA,
        pltpu.SemaphoreType.DMA,
        pltpu.SemaphoreType.DMA,
    ],
)
def _sc_ilm(lp_hbm, y_hbm, zeros_hbm, out_hbm, y_v, idx_v, val_v,
            buf_a, buf_b, sem_a, sem_b, sem_g):
    wid = lax.axis_index("s") * NC + lax.axis_index("c")
    base_row = wid * RPW
    base_flat = base_row * V

    pltpu.sync_copy(y_hbm.at[pl.ds(base_row, RPW)], y_v)
    pltpu.sync_copy(zeros_hbm, buf_a)
    pltpu.sync_copy(zeros_hbm, buf_b)

    for t in range(NG):
        y16 = y_v[pl.ds(t * 16, 16)]
        row16 = lax.iota(jnp.int32, 16) + (base_row + t * 16)
        idx_v[t, pl.ds(0, 16)] = row16 * V + y16
    gh = [
        pltpu.async_copy(lp_hbm.at[idx_v.at[t]], val_v.at[t], sem_g)
        for t in range(NG)
    ]
    for h in gh:
        h.wait()

    for t in range(NG):
        y16 = y_v[pl.ds(t * 16, 16)]
        v16 = val_v[t, pl.ds(0, 16)]
        val_v[t, pl.ds(0, 16)] = jnp.where(
            y16 == 0, jnp.float32(0.0), jnp.float32(1.0 - P) * v16)

    iota16 = lax.iota(jnp.int32, 16)
    zeros16 = jnp.zeros((16,), jnp.float32)
    bufs = (buf_a, buf_b)
    sems = (sem_a, sem_b)
    pending = [None, None]
    dirty = [None, None]
    for c in range(NCHUNK):
        b = c % 2
        buf = bufs[b]
        if pending[b] is not None:
            pending[b].wait()
            for off in dirty[b]:
                buf[pl.ds(off, 16)] = zeros16
        offs = []
        for g in range(GPC):
            t = c * GPC + g
            y16 = y_v[pl.ds(t * 16, 16)]
            v16 = val_v[t, pl.ds(0, 16)]
            for i in range(16):
                y_r = y16[i]
                v_r = v16[i]
                w16 = jnp.where(iota16 == (y_r % 16), v_r, jnp.float32(0.0))
                off = (g * 16 + i) * V + (y_r // 16) * 16
                buf[pl.ds(off, 16)] = w16
                offs.append(off)
        dirty[b] = offs
        pending[b] = pltpu.async_copy(
            buf, out_hbm.at[pl.ds(base_flat + c * ZR * V, ZR * V)], sems[b])
    for b in range(2):
        if pending[b] is not None:
            pending[b].wait()


def kernel(p_fixed, logp_train, y_padded):
    y3 = y_padded.reshape(N, 1, U)
    loss, kl = _tc_call(y3, p_fixed, logp_train)
    zeros = jnp.zeros((ZR * V,), jnp.float32)
    ilm_flat = _sc_ilm(logp_train.reshape(-1), y_padded.reshape(-1), zeros)
    return loss, ilm_flat.reshape(N, U, V), kl


# speedup vs baseline: 2.6530x; 2.6530x over previous
import jax
import jax.numpy as jnp
from jax import lax
from jax.experimental import pallas as pl

N, U, V = 16, 512, 1024
P = 0.5
NB = 2


def _body(y_ref, pf_ref, lp_ref, loss_ref, ilm_ref, kl_ref):
    y = y_ref[:, 0, :]
    lp = lp_ref[...]
    pf = pf_ref[...]
    ycol = jnp.where(y == 0, jnp.int32(-1), y)[:, :, None]
    iota = lax.broadcasted_iota(jnp.int32, (NB, U, V), 2)
    hit = iota == ycol
    half_lp = jnp.float32(P) * lp
    kl = pf * half_lp
    ilm = jnp.where(hit, half_lp, jnp.float32(0.0))
    ilm_ref[...] = ilm
    kl_ref[...] = kl
    loss_ref[...] = ilm + kl


def kernel(p_fixed, logp_train, y_padded):
    y3 = y_padded.reshape(N, 1, U)
    out_shape = jax.ShapeDtypeStruct((N, U, V), jnp.float32)
    return pl.pallas_call(
        _body,
        grid=(N // NB,),
        in_specs=[
            pl.BlockSpec((NB, 1, U), lambda n: (n, 0, 0)),
            pl.BlockSpec((NB, U, V), lambda n: (n, 0, 0)),
            pl.BlockSpec((NB, U, V), lambda n: (n, 0, 0)),
        ],
        out_specs=[
            pl.BlockSpec((NB, U, V), lambda n: (n, 0, 0)),
            pl.BlockSpec((NB, U, V), lambda n: (n, 0, 0)),
            pl.BlockSpec((NB, U, V), lambda n: (n, 0, 0)),
        ],
        out_shape=[out_shape, out_shape, out_shape],
    )(y3, p_fixed, logp_train)


# speedup vs baseline: 2.6594x; 1.0024x over previous
import jax
import jax.numpy as jnp
from jax import lax
from jax.experimental import pallas as pl
from jax.experimental.pallas import tpu as pltpu

N, U, V = 16, 512, 1024
P = 0.5
NB = 2


def _body(y_ref, pf_ref, lp_ref, loss_ref, ilm_ref, kl_ref):
    y = y_ref[:, 0, :]
    lp = lp_ref[...]
    pf = pf_ref[...]
    ycol = jnp.where(y == 0, jnp.int32(-1), y)[:, :, None]
    iota = lax.broadcasted_iota(jnp.int32, (NB, U, V), 2)
    hit = iota == ycol
    half_lp = jnp.float32(P) * lp
    kl = pf * half_lp
    ilm = jnp.where(hit, half_lp, jnp.float32(0.0))
    ilm_ref[...] = ilm
    kl_ref[...] = kl
    loss_ref[...] = ilm + kl


def kernel(p_fixed, logp_train, y_padded):
    y3 = y_padded.reshape(N, 1, U)
    out_shape = jax.ShapeDtypeStruct((N, U, V), jnp.float32)
    return pl.pallas_call(
        _body,
        grid=(N // NB,),
        in_specs=[
            pl.BlockSpec((NB, 1, U), lambda n: (n, 0, 0)),
            pl.BlockSpec((NB, U, V), lambda n: (n, 0, 0)),
            pl.BlockSpec((NB, U, V), lambda n: (n, 0, 0)),
        ],
        out_specs=[
            pl.BlockSpec((NB, U, V), lambda n: (n, 0, 0)),
            pl.BlockSpec((NB, U, V), lambda n: (n, 0, 0)),
            pl.BlockSpec((NB, U, V), lambda n: (n, 0, 0)),
        ],
        out_shape=[out_shape, out_shape, out_shape],
        compiler_params=pltpu.CompilerParams(vmem_limit_bytes=60 * 1024 * 1024),
    )(y3, p_fixed, logp_train)
